# Initial kernel scaffold; baseline (speedup 1.0000x reference)
import jax
import jax.numpy as jnp
from jax.experimental import pallas as pl


def kernel(users, items, user_emb, item_emb, adj_rows, adj_cols, adj_vals):
    raise NotImplementedError("write your pallas kernel here")


# speedup vs baseline: 3.9163x; 3.9163x over previous
import functools

import jax
import jax.numpy as jnp
from jax import lax
from jax.experimental import pallas as pl
from jax.experimental.pallas import tpu as pltpu
from jax.experimental.pallas import tpu_sc as plsc

NUM_USERS = 5000
N_NODES = 10000
D = 128
NNZ = 320000
BATCH = 4096

NC = 2
NS = 16
NW = NC * NS
L = 16

E = 128
NCHUNKS = NNZ // E
KMAX = (NCHUNKS + NW - 1) // NW
RC = 80
NRC = N_NODES // RC
RKMAX = (NRC + NS - 1) // NS

_MESH = plsc.VectorSubcoreMesh(
    core_axis_name="c", subcore_axis_name="s", num_cores=NC, num_subcores=NS)

_f32 = jnp.float32
_i32 = jnp.int32


def _zero_rows(buf, nrows):
    zero = jnp.zeros((L,), _f32)

    def body(j, _):
        for r in range(D // L):
            buf[j, pl.ds(r * L, L)] = zero
        return 0

    lax.fori_loop(0, nrows, body, 0)


def _propagate_body(src_hbm, rows_hbm, cols_hbm, vals_hbm,
                    out0_hbm, out1_hbm,
                    acc, cols_v, rows_v, vals_v, gbuf, sem):
    c = lax.axis_index("c")
    s = lax.axis_index("s")
    wid = c * NS + s

    _zero_rows(gbuf, RC)

    def zchunk(k, _):
        cid = s + NS * k

        @pl.when(cid < NRC)
        def _():
            pltpu.sync_copy(gbuf.at[pl.ds(0, RC)], acc.at[pl.ds(cid * RC, RC)])
        return 0

    lax.fori_loop(0, RKMAX, zchunk, 0)
    plsc.subcore_barrier()

    def chunk(k, _):
        cid = wid + NW * k

        @pl.when(cid < NCHUNKS)
        def _():
            eb = cid * E
            pltpu.sync_copy(cols_hbm.at[pl.ds(eb, E)], cols_v)
            pltpu.sync_copy(rows_hbm.at[pl.ds(eb, E)], rows_v)
            pltpu.sync_copy(vals_hbm.at[pl.ds(eb, E)], vals_v)
            pltpu.async_copy(src_hbm.at[cols_v], gbuf, sem).wait()

            def srow(j, _):
                vv = plsc.load_gather(vals_v, [jnp.full((L,), j, _i32)])
                for r in range(D // L):
                    gbuf[j, pl.ds(r * L, L)] = gbuf[j, pl.ds(r * L, L)] * vv
                return 0

            lax.fori_loop(0, E, srow, 0)
            pltpu.sync_copy(gbuf, acc.at[rows_v], add=True)
        return 0

    lax.fori_loop(0, KMAX, chunk, 0)
    plsc.subcore_barrier()

    def writeout(out_hbm):
        def wchunk(k, _):
            cid = s + NS * k

            @pl.when(cid < NRC)
            def _():
                rb = cid * RC
                pltpu.sync_copy(acc.at[pl.ds(rb, RC)], gbuf.at[pl.ds(0, RC)])
                pltpu.sync_copy(gbuf.at[pl.ds(0, RC)],
                                out_hbm.at[pl.ds(rb, RC)])
            return 0

        lax.fori_loop(0, RKMAX, wchunk, 0)

    @pl.when(c == 0)
    def _():
        writeout(out0_hbm)

    @pl.when(c == 1)
    def _():
        writeout(out1_hbm)


_propagate = pl.kernel(
    _propagate_body,
    out_type=(jax.ShapeDtypeStruct((N_NODES, D), _f32),
              jax.ShapeDtypeStruct((N_NODES, D), _f32)),
    mesh=_MESH,
    compiler_params=pltpu.CompilerParams(needs_layout_passes=False),
    scratch_types=[
        pltpu.VMEM_SHARED((N_NODES, D), _f32),
        pltpu.VMEM((E,), _i32),
        pltpu.VMEM((E,), _i32),
        pltpu.VMEM((E,), _f32),
        pltpu.VMEM((E, D), _f32),
        pltpu.SemaphoreType.DMA,
    ],
)


CR = 80
NCC = N_NODES // CR
CKMAX = (NCC + NW - 1) // NW


def _combine_body(p0_hbm, p1_hbm, out_hbm, b0, b1, sem):
    c = lax.axis_index("c")
    s = lax.axis_index("s")
    wid = c * NS + s

    def chunk(k, _):
        cid = wid + NW * k

        @pl.when(cid < NCC)
        def _():
            rb = cid * CR
            cp0 = pltpu.async_copy(p0_hbm.at[pl.ds(rb, CR)], b0, sem)
            cp1 = pltpu.async_copy(p1_hbm.at[pl.ds(rb, CR)], b1, sem)
            cp0.wait()
            cp1.wait()

            def add_row(j, _):
                for r in range(D // L):
                    b0[j, pl.ds(r * L, L)] = (b0[j, pl.ds(r * L, L)]
                                              + b1[j, pl.ds(r * L, L)])
                return 0

            lax.fori_loop(0, CR, add_row, 0)
            pltpu.sync_copy(b0, out_hbm.at[pl.ds(rb, CR)])
        return 0

    lax.fori_loop(0, CKMAX, chunk, 0)


_combine = pl.kernel(
    _combine_body,
    out_type=jax.ShapeDtypeStruct((N_NODES, D), _f32),
    mesh=_MESH,
    compiler_params=pltpu.CompilerParams(needs_layout_passes=False),
    scratch_types=[
        pltpu.VMEM((CR, D), _f32),
        pltpu.VMEM((CR, D), _f32),
        pltpu.SemaphoreType.DMA,
    ],
)


PB = BATCH // NW


def _finalize_body(users_hbm, items_hbm, e0_hbm, e1_hbm, p0_hbm, p1_hbm,
                   out_hbm, uidx, iidx, ubuf, ibuf, gtmp, outv, sem):
    c = lax.axis_index("c")
    s = lax.axis_index("s")
    wid = c * NS + s
    base = wid * PB

    pltpu.sync_copy(users_hbm.at[pl.ds(base, PB)], uidx)
    pltpu.sync_copy(items_hbm.at[pl.ds(base, PB)], iidx)

    def shift(g, _):
        iidx[pl.ds(g * L, L)] = iidx[pl.ds(g * L, L)] + NUM_USERS
        return 0

    lax.fori_loop(0, PB // L, shift, 0)

    def gather_sum(idx, dst):
        pltpu.async_copy(e0_hbm.at[idx], dst, sem).wait()

        def accum(src_hbm):
            pltpu.async_copy(src_hbm.at[idx], gtmp, sem).wait()

            def add_row(j, _):
                for r in range(D // L):
                    dst[j, pl.ds(r * L, L)] = (dst[j, pl.ds(r * L, L)]
                                               + gtmp[j, pl.ds(r * L, L)])
                return 0

            lax.fori_loop(0, PB, add_row, 0)

        accum(e1_hbm)
        accum(p0_hbm)
        accum(p1_hbm)

    gather_sum(uidx, ubuf)
    gather_sum(iidx, ibuf)

    for g in range(PB // L):
        rowids = lax.iota(_i32, L) + g * L

        def dstep(dd, acc):
            di = jnp.full((L,), dd, _i32)
            cu = plsc.load_gather(ubuf, [rowids, di])
            ci = plsc.load_gather(ibuf, [rowids, di])
            return acc + cu * ci

        accv = lax.fori_loop(0, D, dstep, jnp.zeros((L,), _f32))
        outv[pl.ds(g * L, L)] = accv * (1.0 / 9.0)

    pltpu.sync_copy(outv, out_hbm.at[pl.ds(base, PB)])


_finalize = pl.kernel(
    _finalize_body,
    out_type=jax.ShapeDtypeStruct((BATCH,), _f32),
    mesh=_MESH,
    compiler_params=pltpu.CompilerParams(needs_layout_passes=False),
    scratch_types=[
        pltpu.VMEM((PB,), _i32),
        pltpu.VMEM((PB,), _i32),
        pltpu.VMEM((PB, D), _f32),
        pltpu.VMEM((PB, D), _f32),
        pltpu.VMEM((PB, D), _f32),
        pltpu.VMEM((PB,), _f32),
        pltpu.SemaphoreType.DMA,
    ],
)


@jax.jit
def kernel(users, items, user_emb, item_emb, adj_rows, adj_cols, adj_vals):
    ego0 = jnp.concatenate([user_emb, item_emb], axis=0)
    p10, p11 = _propagate(ego0, adj_rows, adj_cols, adj_vals)
    ego1 = _combine(p10, p11)
    p20, p21 = _propagate(ego1, adj_rows, adj_cols, adj_vals)
    return _finalize(users.astype(_i32), items.astype(_i32),
                     ego0, ego1, p20, p21)
